# baseline (speedup 1.0000x reference)
import functools

import jax
import jax.numpy as jnp
from jax import lax
from jax.experimental import pallas as pl
from jax.experimental.pallas import tpu as pltpu
from jax.experimental.pallas import tpu_sc as plsc


@functools.lru_cache(maxsize=None)
def _make_sc_gather(V, D, B):
    info = plsc.get_sparse_core_info()
    NC, NS, L = info.num_cores, info.num_subcores, info.num_lanes
    NW = NC * NS
    assert D % L == 0 and B % (8 * NW) == 0
    b_per_w = B // NW
    mesh = plsc.VectorSubcoreMesh(core_axis_name="c", subcore_axis_name="s")

    @functools.partial(
        pl.kernel,
        mesh=mesh,
        out_type=jax.ShapeDtypeStruct((B, D), jnp.float32),
        scratch_types=[
            pltpu.VMEM((b_per_w,), jnp.int32),
            pltpu.VMEM((b_per_w, D), jnp.float32),
            pltpu.SemaphoreType.DMA,
        ],
        compiler_params=pltpu.CompilerParams(use_tc_tiling_on_sc=False),
    )
    def sc_gather(table_hbm, idx_hbm, out_hbm, idx_v, rows_v, sem):
        wid = lax.axis_index("s") * NC + lax.axis_index("c")
        base = wid * b_per_w
        pltpu.sync_copy(idx_hbm.at[pl.ds(base, b_per_w)], idx_v)
        pltpu.async_copy(table_hbm.at[idx_v], rows_v, sem).wait()
        pltpu.sync_copy(rows_v, out_hbm.at[pl.ds(base, b_per_w)])

    return sc_gather


_TN = (((0,), (0,)), ((), ()))


def _tc_body(hid_ref, wtb_ref, out_ref, m_ref, s_ref, hidt_ref, *,
             BT, NV, NB, VC, DK):
    i = pl.program_id(0)
    j = pl.program_id(1)
    a = i % 2

    @pl.when(j == 0)
    def _init():
        hidt_ref[a] = hid_ref[...].T

    l = lax.dot_general(wtb_ref[...], hidt_ref[a], _TN,
                        preferred_element_type=jnp.float32)
    out_ref[...] = lax.dot_general(wtb_ref[...], hidt_ref[1 - a], _TN,
                                   preferred_element_type=jnp.float32)

    @pl.when(j == 0)
    def _prep():
        m_ref[...] = jnp.max(l, axis=0, keepdims=True)
        s_ref[...] = jnp.zeros((1, BT), jnp.float32)

    e = jnp.exp(l - m_ref[...])
    s_ref[...] += jnp.sum(e, axis=0, keepdims=True)

    @pl.when(j == NV - 1)
    def _fin():
        hidt_ref[a, DK - 1 : DK, :] = -(m_ref[...] + jnp.log(s_ref[...]))


@functools.lru_cache(maxsize=None)
def _make_tc_logsoftmax(B, V, D, BT, VC):
    NV = -(-V // VC)
    NB = B // BT
    DK = D + 2
    body = functools.partial(_tc_body, BT=BT, NV=NV, NB=NB, VC=VC, DK=DK)
    return pl.pallas_call(
        body,
        grid=(NB + 1, NV),
        in_specs=[
            pl.BlockSpec((BT, DK), lambda i, j: (jnp.minimum(i, NB - 1), 0)),
            pl.BlockSpec((DK, VC), lambda i, j: (0, j)),
        ],
        out_specs=pl.BlockSpec(
            (VC, BT),
            lambda i, j: (jnp.where(i == 0, 0, j), jnp.maximum(i - 1, 0)),
        ),
        out_shape=jax.ShapeDtypeStruct((V, B), jnp.float32),
        scratch_shapes=[
            pltpu.VMEM((1, BT), jnp.float32),
            pltpu.VMEM((1, BT), jnp.float32),
            pltpu.VMEM((2, DK, BT), jnp.float32),
        ],
    )


def kernel(center_word_index, emb_table, W, b):
    V, D = emb_table.shape
    (B,) = center_word_index.shape
    BT = 512
    VC = 4096
    NV = -(-V // VC)
    VPAD = NV * VC

    idx = center_word_index.astype(jnp.int32)
    hidden = _make_sc_gather(V, D, B)(emb_table, idx)

    wt_p = jnp.pad(W.T, ((0, 0), (0, VPAD - V)))
    b_p = jnp.pad(b, (0, VPAD - V), constant_values=-1e30)
    wtb = jnp.concatenate(
        [wt_p, b_p[None, :], jnp.ones((1, VPAD), jnp.float32)], axis=0
    )
    hid1 = jnp.concatenate(
        [hidden, jnp.ones((B, 1), jnp.float32), jnp.zeros((B, 1), jnp.float32)],
        axis=1,
    )

    out_t = _make_tc_logsoftmax(B, V, D, BT, VC)(hid1, wtb)
    return out_t.T


# speedup vs baseline: 1.0810x; 1.0810x over previous
import functools

import jax
import jax.numpy as jnp
from jax import lax
from jax.experimental import pallas as pl
from jax.experimental.pallas import tpu as pltpu
from jax.experimental.pallas import tpu_sc as plsc


@functools.lru_cache(maxsize=None)
def _make_sc_gather(V, D, B):
    info = plsc.get_sparse_core_info()
    NC, NS, L = info.num_cores, info.num_subcores, info.num_lanes
    NW = NC * NS
    assert D % L == 0 and B % (8 * NW) == 0
    b_per_w = B // NW
    mesh = plsc.VectorSubcoreMesh(core_axis_name="c", subcore_axis_name="s")

    @functools.partial(
        pl.kernel,
        mesh=mesh,
        out_type=jax.ShapeDtypeStruct((B, D), jnp.float32),
        scratch_types=[
            pltpu.VMEM((b_per_w,), jnp.int32),
            pltpu.VMEM((b_per_w, D), jnp.float32),
            pltpu.SemaphoreType.DMA,
        ],
        compiler_params=pltpu.CompilerParams(use_tc_tiling_on_sc=False),
    )
    def sc_gather(table_hbm, idx_hbm, out_hbm, idx_v, rows_v, sem):
        wid = lax.axis_index("s") * NC + lax.axis_index("c")
        base = wid * b_per_w
        pltpu.sync_copy(idx_hbm.at[pl.ds(base, b_per_w)], idx_v)
        pltpu.async_copy(table_hbm.at[idx_v], rows_v, sem).wait()
        pltpu.sync_copy(rows_v, out_hbm.at[pl.ds(base, b_per_w)])

    return sc_gather


_TN = (((0,), (0,)), ((), ()))


def _tc_body(hid_ref, wtb_ref, out_ref, m_ref, s_ref, hidt_ref, *,
             BT, NV, NB, VC, DK):
    i = pl.program_id(0)
    j = pl.program_id(1)
    a = i % 2

    @pl.when(j == 0)
    def _init():
        hidt_ref[a] = hid_ref[...].T
        m_ref[...] = jnp.full((1, BT), -1e30, jnp.float32)
        s_ref[...] = jnp.zeros((1, BT), jnp.float32)

    l = lax.dot_general(wtb_ref[...], hidt_ref[a], _TN,
                        preferred_element_type=jnp.float32)
    out_ref[...] = lax.dot_general(wtb_ref[...], hidt_ref[1 - a], _TN,
                                   preferred_element_type=jnp.float32)
    m = m_ref[...]
    m2 = jnp.maximum(m, jnp.max(l, axis=0, keepdims=True))
    s_ref[...] = s_ref[...] * jnp.exp(m - m2) + jnp.sum(
        jnp.exp(l - m2), axis=0, keepdims=True
    )
    m_ref[...] = m2

    @pl.when(j == NV - 1)
    def _fin():
        hidt_ref[a, DK - 1 : DK, :] = -(m_ref[...] + jnp.log(s_ref[...]))


@functools.lru_cache(maxsize=None)
def _make_tc_logsoftmax(B, V, D, BT, VC):
    NV = -(-V // VC)
    NB = B // BT
    DK = D + 2
    body = functools.partial(_tc_body, BT=BT, NV=NV, NB=NB, VC=VC, DK=DK)
    return pl.pallas_call(
        body,
        grid=(NB + 1, NV),
        in_specs=[
            pl.BlockSpec((BT, DK), lambda i, j: (jnp.minimum(i, NB - 1), 0)),
            pl.BlockSpec((DK, VC), lambda i, j: (0, j)),
        ],
        out_specs=pl.BlockSpec(
            (VC, BT),
            lambda i, j: (jnp.where(i == 0, 0, j), jnp.maximum(i - 1, 0)),
        ),
        out_shape=jax.ShapeDtypeStruct((V, B), jnp.float32),
        scratch_shapes=[
            pltpu.VMEM((1, BT), jnp.float32),
            pltpu.VMEM((1, BT), jnp.float32),
            pltpu.VMEM((2, DK, BT), jnp.float32),
        ],
    )


def kernel(center_word_index, emb_table, W, b):
    V, D = emb_table.shape
    (B,) = center_word_index.shape
    BT = 512
    VC = 4096
    NV = -(-V // VC)
    VPAD = NV * VC

    idx = center_word_index.astype(jnp.int32)
    hidden = _make_sc_gather(V, D, B)(emb_table, idx)

    wt_p = jnp.pad(W.T, ((0, 0), (0, VPAD - V)))
    b_p = jnp.pad(b, (0, VPAD - V), constant_values=-1e30)
    wtb = jnp.concatenate(
        [wt_p, b_p[None, :], jnp.ones((1, VPAD), jnp.float32)], axis=0
    )
    hid1 = jnp.concatenate(
        [hidden, jnp.ones((B, 1), jnp.float32), jnp.zeros((B, 1), jnp.float32)],
        axis=1,
    )

    out_t = _make_tc_logsoftmax(B, V, D, BT, VC)(hid1, wtb)
    return out_t.T
